# baseline (speedup 1.0000x reference)
import functools

import jax
import jax.numpy as jnp
from jax import lax
from jax.experimental import pallas as pl
from jax.experimental.pallas import tpu as pltpu
from jax.experimental.pallas import tpu_sc as plsc

SEL = 4096
LANES = 16
RADIX = 256


def _keys_body(n, cls_ref, keys_ref):
  m = jnp.max(cls_ref[...], axis=0)
  bits = lax.bitcast_convert_type(m, jnp.int32)
  ck = jnp.where(bits < 0, bits, ~(bits | jnp.int32(-2147483648)))
  keys_ref[:, :n] = ck
  keys_ref[:, n:] = jnp.full(
      (keys_ref.shape[0], keys_ref.shape[1] - n), -1, jnp.int32)


def _sc_body(n, npad, keys_hbm, box_hbm, cls_hbm, obox_hbm, ocls_hbm,
             keys_a, idx_a, keys_b, idx_b, hist, idx2d, rows7, rows3, sem):
  b = lax.axis_index("s") * 2 + lax.axis_index("c")
  nb = keys_hbm.shape[0]
  chunk = npad // LANES
  iota = lax.iota(jnp.int32, LANES)
  lane_base = iota * chunk
  ones = jnp.ones((LANES,), jnp.int32)

  @pl.when(b < nb)
  def _():
    pltpu.sync_copy(keys_hbm.at[b], keys_a)

    base = b * n

    @pl.loop(0, npad // LANES)
    def _(t):
      idx_a[pl.ds(t * LANES, LANES)] = t * LANES + iota + base

    for p, (ks, vs, kd, vd) in enumerate((
        (keys_a, idx_a, keys_b, idx_b),
        (keys_b, idx_b, keys_a, idx_a),
        (keys_a, idx_a, keys_b, idx_b),
        (keys_b, idx_b, None, None),
    )):
      shift = 8 * p

      @pl.loop(0, RADIX)
      def _(d):
        hist[d] = jnp.zeros((LANES,), jnp.int32)

      @pl.loop(0, chunk)
      def _(t):
        k = plsc.load_gather(ks, [lane_base + t])
        d = lax.shift_right_logical(k, shift) & 0xFF
        plsc.addupdate_scatter(hist, [d, iota], ones)

      @pl.loop(0, RADIX, init_carry=jnp.int32(0))
      def _(d, run):
        v = hist[d]
        inc = plsc.cumsum(v)
        hist[d] = (inc - v) + run
        return run + jnp.sum(v)

      if kd is not None:
        @pl.loop(0, chunk)
        def _(t):
          g = lane_base + t
          k = plsc.load_gather(ks, [g])
          v = plsc.load_gather(vs, [g])
          d = lax.shift_right_logical(k, shift) & 0xFF
          pos = plsc.load_gather(hist, [d, iota])
          plsc.store_scatter(kd, [pos], k)
          plsc.store_scatter(vd, [pos], v)
          plsc.store_scatter(hist, [d, iota], pos + 1)
      else:
        @pl.loop(0, chunk)
        def _(t):
          g = lane_base + t
          k = plsc.load_gather(ks, [g])
          v = plsc.load_gather(vs, [g])
          d = lax.shift_right_logical(k, shift) & 0xFF
          pos = plsc.load_gather(hist, [d, iota])
          plsc.store_scatter(idx2d, [lax.shift_right_logical(pos, 7),
                                     pos & 127], v, mask=pos < SEL)
          plsc.store_scatter(hist, [d, iota], pos + 1)

    for q in range(SEL // 512):
      descs = []
      for j in range(4):
        r = 4 * q + j
        descs.append(pltpu.async_copy(
            box_hbm.at[idx2d.at[r]], rows7.at[pl.ds(j * 128, 128)], sem))
        descs.append(pltpu.async_copy(
            cls_hbm.at[idx2d.at[r]], rows3.at[pl.ds(j * 128, 128)], sem))
      for dsc in descs:
        dsc.wait()
      pltpu.sync_copy(rows7, obox_hbm.at[pl.ds(b * SEL + q * 512, 512)])
      pltpu.sync_copy(rows3, ocls_hbm.at[pl.ds(b * SEL + q * 512, 512)])


def _mm(x, w):
  return lax.dot_general(x, w, (((1,), (0,)), ((), ())),
                         preferred_element_type=jnp.float32)


def _head_body(box_ref, cls_ref, w1_ref, b1_ref, w2_ref, b2_ref, wb_ref,
               bb_ref, wr_ref, br_ref, bin_ref, res_ref):
  x = jnp.concatenate([box_ref[0], cls_ref[0]], axis=1)
  z = jnp.zeros((1, x.shape[1]), jnp.float32)
  xd = jnp.concatenate([z, x[:-1, :]], axis=0)
  xu = jnp.concatenate([x[1:, :], z], axis=0)
  w1 = w1_ref[...]
  h1 = _mm(xd, w1[0:16]) + _mm(x, w1[16:32]) + _mm(xu, w1[32:48])
  h1 = jnp.maximum(h1 + b1_ref[...], 0.0)
  z1 = jnp.zeros((1, h1.shape[1]), jnp.float32)
  h1d = jnp.concatenate([z1, h1[:-1, :]], axis=0)
  h1u = jnp.concatenate([h1[1:, :], z1], axis=0)
  w2 = w2_ref[...]
  h2 = _mm(h1d, w2[0:32]) + _mm(h1, w2[32:64]) + _mm(h1u, w2[64:96])
  h2 = jnp.maximum(h2 + b2_ref[...], 0.0)
  bin_ref[0] = _mm(h2, wb_ref[...]) + bb_ref[...]
  res_ref[0] = _mm(h2, wr_ref[...]) + br_ref[...]


def kernel(rpn_box_preds, rpn_cls_preds, batch_size, w1, g1, be1, rm1, rv1,
           w2, g2, be2, rm2, rv2, wb, bb, wr, br):
  bsz, n, _ = rpn_box_preds.shape
  npad = ((n + 2047) // 2048) * 2048

  box2d = jnp.reshape(jnp.concatenate(
      [rpn_box_preds, jnp.zeros((bsz, n, 1), jnp.float32)], axis=-1),
      (bsz * n, 8))
  cls2d = jnp.reshape(jnp.concatenate(
      [rpn_cls_preds, jnp.zeros((bsz, n, 5), jnp.float32)], axis=-1),
      (bsz * n, 8))
  cls_t = jnp.transpose(rpn_cls_preds, (2, 0, 1))

  eps = 1e-5
  s1 = g1 * lax.rsqrt(rv1 + eps)
  wt1 = jnp.transpose(w1 * s1[:, None, None], (2, 1, 0))
  w1c = jnp.zeros((3, 16, 32), jnp.float32)
  w1c = w1c.at[:, 0:7].set(wt1[:, 0:7]).at[:, 8:11].set(wt1[:, 7:10])
  w1c = w1c.reshape(48, 32)
  b1c = be1 - rm1 * s1
  s2 = g2 * lax.rsqrt(rv2 + eps)
  w2c = jnp.transpose(w2 * s2[:, None, None], (2, 1, 0)).reshape(96, 64)
  b2c = be2 - rm2 * s2
  wb2 = wb[:, :, 0].T
  wr2 = wr[:, :, 0].T

  keys = pl.pallas_call(
      functools.partial(_keys_body, n),
      out_shape=jax.ShapeDtypeStruct((bsz, npad), jnp.int32),
  )(cls_t)

  mesh = plsc.VectorSubcoreMesh(core_axis_name="c", subcore_axis_name="s",
                                num_cores=2, num_subcores=16)
  box_sel2, cls_sel2 = pl.kernel(
      functools.partial(_sc_body, n, npad),
      out_type=[
          jax.ShapeDtypeStruct((bsz * SEL, 8), jnp.float32),
          jax.ShapeDtypeStruct((bsz * SEL, 8), jnp.float32),
      ],
      mesh=mesh,
      compiler_params=pltpu.CompilerParams(needs_layout_passes=False,
                                           use_tc_tiling_on_sc=False),
      scratch_types=[
          pltpu.VMEM((npad,), jnp.int32),
          pltpu.VMEM((npad,), jnp.int32),
          pltpu.VMEM((npad,), jnp.int32),
          pltpu.VMEM((npad,), jnp.int32),
          pltpu.VMEM((RADIX, LANES), jnp.int32),
          pltpu.VMEM((32, 128), jnp.int32),
          pltpu.VMEM((512, 8), jnp.float32),
          pltpu.VMEM((512, 8), jnp.float32),
          pltpu.SemaphoreType.DMA,
      ],
  )(keys, box2d, cls2d)

  box_sel8 = jnp.reshape(box_sel2, (bsz, SEL, 8))
  cls_sel8 = jnp.reshape(cls_sel2, (bsz, SEL, 8))
  box_sel = box_sel8[:, :, 0:7]
  cls_sel = cls_sel8[:, :, 0:3]

  iou_bin, iou_res = pl.pallas_call(
      _head_body,
      grid=(bsz,),
      in_specs=[
          pl.BlockSpec((1, SEL, 8), lambda i: (i, 0, 0)),
          pl.BlockSpec((1, SEL, 8), lambda i: (i, 0, 0)),
          pl.BlockSpec((48, 32), lambda i: (0, 0)),
          pl.BlockSpec((32,), lambda i: (0,)),
          pl.BlockSpec((96, 64), lambda i: (0, 0)),
          pl.BlockSpec((64,), lambda i: (0,)),
          pl.BlockSpec((64, 5), lambda i: (0, 0)),
          pl.BlockSpec((5,), lambda i: (0,)),
          pl.BlockSpec((64, 1), lambda i: (0, 0)),
          pl.BlockSpec((1,), lambda i: (0,)),
      ],
      out_specs=[
          pl.BlockSpec((1, SEL, 5), lambda i: (i, 0, 0)),
          pl.BlockSpec((1, SEL, 1), lambda i: (i, 0, 0)),
      ],
      out_shape=[
          jax.ShapeDtypeStruct((bsz, SEL, 5), jnp.float32),
          jax.ShapeDtypeStruct((bsz, SEL, 1), jnp.float32),
      ],
  )(box_sel8, cls_sel8, w1c, b1c, w2c, b2c, wb2, bb, wr2, br)

  return (iou_bin, iou_res, box_sel, cls_sel)


# speedup vs baseline: 1.3441x; 1.3441x over previous
import functools

import jax
import jax.numpy as jnp
from jax import lax
from jax.experimental import pallas as pl
from jax.experimental.pallas import tpu as pltpu
from jax.experimental.pallas import tpu_sc as plsc

SEL = 4096
LANES = 16
RADIX = 256


def _pack_body(box_ref, cls_ref, comb_ref):
  box = box_ref[0]
  cls = cls_ref[0]
  nc = box.shape[0]
  comb_ref[0] = jnp.concatenate(
      [box, jnp.zeros((nc, 1), jnp.float32), cls,
       jnp.zeros((nc, 5), jnp.float32)], axis=1)


def _keys_body(n, cls_ref, keys_ref):
  m = jnp.max(cls_ref[...], axis=0)
  bits = lax.bitcast_convert_type(m, jnp.int32)
  ck = jnp.where(bits < 0, bits, ~(bits | jnp.int32(-2147483648)))
  keys_ref[:, :n] = ck
  keys_ref[:, n:] = jnp.full(
      (keys_ref.shape[0], keys_ref.shape[1] - n), -1, jnp.int32)


def _sc_body(n, npad, keys_hbm, comb_hbm, out_hbm,
             keys_a, idx_a, keys_b, idx_b, hist, idx2d, rows, sem):
  b = lax.axis_index("s") * 2 + lax.axis_index("c")
  nb = keys_hbm.shape[0]
  chunk = npad // LANES
  iota = lax.iota(jnp.int32, LANES)
  lane_base = iota * chunk
  ones = jnp.ones((LANES,), jnp.int32)

  @pl.when(b < nb)
  def _():
    pltpu.sync_copy(keys_hbm.at[b], keys_a)

    @pl.loop(0, npad // LANES)
    def _(t):
      idx_a[pl.ds(t * LANES, LANES)] = t * LANES + iota

    for p, (ks, vs, kd, vd) in enumerate((
        (keys_a, idx_a, keys_b, idx_b),
        (keys_b, idx_b, keys_a, idx_a),
        (keys_a, idx_a, keys_b, idx_b),
        (keys_b, idx_b, None, None),
    )):
      shift = 8 * p

      @pl.loop(0, RADIX)
      def _(d):
        hist[d] = jnp.zeros((LANES,), jnp.int32)

      @pl.loop(0, chunk)
      def _(t):
        k = plsc.load_gather(ks, [lane_base + t])
        d = lax.shift_right_logical(k, shift) & 0xFF
        plsc.addupdate_scatter(hist, [d, iota], ones)

      @pl.loop(0, RADIX, init_carry=jnp.int32(0))
      def _(d, run):
        v = hist[d]
        inc = plsc.cumsum(v)
        hist[d] = (inc - v) + run
        return run + jnp.sum(v)

      if kd is not None:
        @pl.loop(0, chunk)
        def _(t):
          g = lane_base + t
          k = plsc.load_gather(ks, [g])
          v = plsc.load_gather(vs, [g])
          d = lax.shift_right_logical(k, shift) & 0xFF
          pos = plsc.load_gather(hist, [d, iota])
          plsc.store_scatter(kd, [pos], k)
          plsc.store_scatter(vd, [pos], v)
          plsc.store_scatter(hist, [d, iota], pos + 1)
      else:
        @pl.loop(0, chunk)
        def _(t):
          g = lane_base + t
          k = plsc.load_gather(ks, [g])
          v = plsc.load_gather(vs, [g])
          d = lax.shift_right_logical(k, shift) & 0xFF
          pos = plsc.load_gather(hist, [d, iota])
          plsc.store_scatter(idx2d, [lax.shift_right_logical(pos, 7),
                                     pos & 127], v, mask=pos < SEL)
          plsc.store_scatter(hist, [d, iota], pos + 1)

    for q in range(2):
      descs = []
      for j in range(16):
        descs.append(pltpu.async_copy(
            comb_hbm.at[b].at[idx2d.at[16 * q + j]],
            rows.at[pl.ds(j * 128, 128)], sem))
      for dsc in descs:
        dsc.wait()
      pltpu.sync_copy(rows, out_hbm.at[b].at[pl.ds(q * 2048, 2048)])


def _mm(x, w):
  return lax.dot_general(x, w, (((1,), (0,)), ((), ())),
                         preferred_element_type=jnp.float32)


def _head_body(comb_ref, w1_ref, b1_ref, w2_ref, b2_ref, wb_ref, bb_ref,
               wr_ref, br_ref, bin_ref, res_ref):
  x = comb_ref[0]
  z = jnp.zeros((1, x.shape[1]), jnp.float32)
  xd = jnp.concatenate([z, x[:-1, :]], axis=0)
  xu = jnp.concatenate([x[1:, :], z], axis=0)
  w1 = w1_ref[...]
  h1 = _mm(xd, w1[0:16]) + _mm(x, w1[16:32]) + _mm(xu, w1[32:48])
  h1 = jnp.maximum(h1 + b1_ref[...], 0.0)
  z1 = jnp.zeros((1, h1.shape[1]), jnp.float32)
  h1d = jnp.concatenate([z1, h1[:-1, :]], axis=0)
  h1u = jnp.concatenate([h1[1:, :], z1], axis=0)
  w2 = w2_ref[...]
  h2 = _mm(h1d, w2[0:32]) + _mm(h1, w2[32:64]) + _mm(h1u, w2[64:96])
  h2 = jnp.maximum(h2 + b2_ref[...], 0.0)
  bin_ref[0] = _mm(h2, wb_ref[...]) + bb_ref[...]
  res_ref[0] = _mm(h2, wr_ref[...]) + br_ref[...]


def kernel(rpn_box_preds, rpn_cls_preds, batch_size, w1, g1, be1, rm1, rv1,
           w2, g2, be2, rm2, rv2, wb, bb, wr, br):
  bsz, n, _ = rpn_box_preds.shape
  npad = ((n + 127) // 128) * 128

  cls_t = jnp.transpose(rpn_cls_preds, (2, 0, 1))

  eps = 1e-5
  s1 = g1 * lax.rsqrt(rv1 + eps)
  wt1 = jnp.transpose(w1 * s1[:, None, None], (2, 1, 0))
  w1c = jnp.zeros((3, 16, 32), jnp.float32)
  w1c = w1c.at[:, 0:7].set(wt1[:, 0:7]).at[:, 8:11].set(wt1[:, 7:10])
  w1c = w1c.reshape(48, 32)
  b1c = be1 - rm1 * s1
  s2 = g2 * lax.rsqrt(rv2 + eps)
  w2c = jnp.transpose(w2 * s2[:, None, None], (2, 1, 0)).reshape(96, 64)
  b2c = be2 - rm2 * s2
  wb2 = wb[:, :, 0].T
  wr2 = wr[:, :, 0].T

  nc = 2000
  comb = pl.pallas_call(
      _pack_body,
      grid=(bsz, n // nc),
      in_specs=[
          pl.BlockSpec((1, nc, 7), lambda b, c: (b, c, 0)),
          pl.BlockSpec((1, nc, 3), lambda b, c: (b, c, 0)),
      ],
      out_specs=pl.BlockSpec((1, nc, 16), lambda b, c: (b, c, 0)),
      out_shape=jax.ShapeDtypeStruct((bsz, n, 16), jnp.float32),
  )(rpn_box_preds, rpn_cls_preds)

  keys = pl.pallas_call(
      functools.partial(_keys_body, n),
      out_shape=jax.ShapeDtypeStruct((bsz, npad), jnp.int32),
  )(cls_t)

  mesh = plsc.VectorSubcoreMesh(core_axis_name="c", subcore_axis_name="s",
                                num_cores=2, num_subcores=16)
  comb_sel = pl.kernel(
      functools.partial(_sc_body, n, npad),
      out_type=jax.ShapeDtypeStruct((bsz, SEL, 16), jnp.float32),
      mesh=mesh,
      compiler_params=pltpu.CompilerParams(needs_layout_passes=False,
                                           use_tc_tiling_on_sc=False),
      scratch_types=[
          pltpu.VMEM((npad,), jnp.int32),
          pltpu.VMEM((npad,), jnp.int32),
          pltpu.VMEM((npad,), jnp.int32),
          pltpu.VMEM((npad,), jnp.int32),
          pltpu.VMEM((RADIX, LANES), jnp.int32),
          pltpu.VMEM((32, 128), jnp.int32),
          pltpu.VMEM((2048, 16), jnp.float32),
          pltpu.SemaphoreType.DMA,
      ],
  )(keys, comb)

  iou_bin, iou_res = pl.pallas_call(
      _head_body,
      grid=(bsz,),
      in_specs=[
          pl.BlockSpec((1, SEL, 16), lambda i: (i, 0, 0)),
          pl.BlockSpec((48, 32), lambda i: (0, 0)),
          pl.BlockSpec((32,), lambda i: (0,)),
          pl.BlockSpec((96, 64), lambda i: (0, 0)),
          pl.BlockSpec((64,), lambda i: (0,)),
          pl.BlockSpec((64, 5), lambda i: (0, 0)),
          pl.BlockSpec((5,), lambda i: (0,)),
          pl.BlockSpec((64, 1), lambda i: (0, 0)),
          pl.BlockSpec((1,), lambda i: (0,)),
      ],
      out_specs=[
          pl.BlockSpec((1, SEL, 5), lambda i: (i, 0, 0)),
          pl.BlockSpec((1, SEL, 1), lambda i: (i, 0, 0)),
      ],
      out_shape=[
          jax.ShapeDtypeStruct((bsz, SEL, 5), jnp.float32),
          jax.ShapeDtypeStruct((bsz, SEL, 1), jnp.float32),
      ],
  )(comb_sel, w1c, b1c, w2c, b2c, wb2, bb, wr2, br)

  box_sel = comb_sel[:, :, 0:7]
  cls_sel = comb_sel[:, :, 8:11]
  return (iou_bin, iou_res, box_sel, cls_sel)


# speedup vs baseline: 1.3794x; 1.0262x over previous
import functools

import jax
import jax.numpy as jnp
from jax import lax
from jax.experimental import pallas as pl
from jax.experimental.pallas import tpu as pltpu
from jax.experimental.pallas import tpu_sc as plsc

SEL = 4096
LANES = 16
RADIX = 256


def _pack_body(box_ref, cls_ref, comb_ref):
  box = box_ref[0]
  cls = cls_ref[0]
  nc = box.shape[0]
  comb_ref[0] = jnp.concatenate(
      [box, jnp.zeros((nc, 1), jnp.float32), cls,
       jnp.zeros((nc, 5), jnp.float32)], axis=1)


def _keys_body(n, cls_ref, keys_ref):
  m = jnp.max(cls_ref[...], axis=0)
  bits = lax.bitcast_convert_type(m, jnp.int32)
  ck = jnp.where(bits < 0, bits, ~(bits | jnp.int32(-2147483648)))
  keys_ref[:, :n] = ck
  keys_ref[:, n:] = jnp.full(
      (keys_ref.shape[0], keys_ref.shape[1] - n), -1, jnp.int32)


def _sc_body(n, npad, keys_hbm, comb_hbm, out_hbm,
             keys_a, idx_a, keys_b, idx_b, hist, idx2d, rows, sem):
  b = lax.axis_index("s") * 2 + lax.axis_index("c")
  nb = keys_hbm.shape[0]
  nstream = 4
  vl = nstream * LANES
  chunk = npad // vl
  iota = lax.iota(jnp.int32, LANES)
  lane_bases = [(s * LANES + iota) * chunk for s in range(nstream)]
  lane_cols = [s * LANES + iota for s in range(nstream)]
  ones = jnp.ones((LANES,), jnp.int32)

  @pl.when(b < nb)
  def _():
    pltpu.sync_copy(keys_hbm.at[b], keys_a)

    @pl.loop(0, npad // LANES)
    def _(t):
      idx_a[pl.ds(t * LANES, LANES)] = t * LANES + iota

    for p, (ks, vs, kd, vd) in enumerate((
        (keys_a, idx_a, keys_b, idx_b),
        (keys_b, idx_b, keys_a, idx_a),
        (keys_a, idx_a, keys_b, idx_b),
        (keys_b, idx_b, None, None),
    )):
      shift = 8 * p

      @pl.loop(0, RADIX)
      def _(d):
        for s in range(nstream):
          hist[d, pl.ds(s * LANES, LANES)] = jnp.zeros((LANES,), jnp.int32)

      @pl.loop(0, chunk)
      def _(t):
        for s in range(nstream):
          k = plsc.load_gather(ks, [lane_bases[s] + t])
          d = lax.shift_right_logical(k, shift) & 0xFF
          plsc.addupdate_scatter(hist, [d, lane_cols[s]], ones)

      @pl.loop(0, RADIX, init_carry=jnp.int32(0))
      def _(d, run):
        for s in range(nstream):
          v = hist[d, pl.ds(s * LANES, LANES)]
          inc = plsc.cumsum(v)
          hist[d, pl.ds(s * LANES, LANES)] = (inc - v) + run
          run = run + jnp.sum(v)
        return run

      if kd is not None:
        @pl.loop(0, chunk)
        def _(t):
          for s in range(nstream):
            g = lane_bases[s] + t
            k = plsc.load_gather(ks, [g])
            v = plsc.load_gather(vs, [g])
            d = lax.shift_right_logical(k, shift) & 0xFF
            pos = plsc.load_gather(hist, [d, lane_cols[s]])
            plsc.store_scatter(kd, [pos], k)
            plsc.store_scatter(vd, [pos], v)
            plsc.store_scatter(hist, [d, lane_cols[s]], pos + 1)
      else:
        @pl.loop(0, chunk)
        def _(t):
          for s in range(nstream):
            g = lane_bases[s] + t
            k = plsc.load_gather(ks, [g])
            v = plsc.load_gather(vs, [g])
            d = lax.shift_right_logical(k, shift) & 0xFF
            pos = plsc.load_gather(hist, [d, lane_cols[s]])
            plsc.store_scatter(idx2d, [lax.shift_right_logical(pos, 7),
                                       pos & 127], v, mask=pos < SEL)
            plsc.store_scatter(hist, [d, lane_cols[s]], pos + 1)

    for q in range(4):
      descs = []
      for j in range(8):
        descs.append(pltpu.async_copy(
            comb_hbm.at[b].at[idx2d.at[8 * q + j]],
            rows.at[pl.ds(j * 128, 128)], sem))
      for dsc in descs:
        dsc.wait()
      pltpu.sync_copy(rows, out_hbm.at[b].at[pl.ds(q * 1024, 1024)])


def _mm(x, w):
  return lax.dot_general(x, w, (((1,), (0,)), ((), ())),
                         preferred_element_type=jnp.float32)


def _head_body(comb_ref, w1_ref, b1_ref, w2_ref, b2_ref, wb_ref, bb_ref,
               wr_ref, br_ref, bin_ref, res_ref):
  x = comb_ref[0]
  z = jnp.zeros((1, x.shape[1]), jnp.float32)
  xd = jnp.concatenate([z, x[:-1, :]], axis=0)
  xu = jnp.concatenate([x[1:, :], z], axis=0)
  w1 = w1_ref[...]
  h1 = _mm(xd, w1[0:16]) + _mm(x, w1[16:32]) + _mm(xu, w1[32:48])
  h1 = jnp.maximum(h1 + b1_ref[...], 0.0)
  z1 = jnp.zeros((1, h1.shape[1]), jnp.float32)
  h1d = jnp.concatenate([z1, h1[:-1, :]], axis=0)
  h1u = jnp.concatenate([h1[1:, :], z1], axis=0)
  w2 = w2_ref[...]
  h2 = _mm(h1d, w2[0:32]) + _mm(h1, w2[32:64]) + _mm(h1u, w2[64:96])
  h2 = jnp.maximum(h2 + b2_ref[...], 0.0)
  bin_ref[0] = _mm(h2, wb_ref[...]) + bb_ref[...]
  res_ref[0] = _mm(h2, wr_ref[...]) + br_ref[...]


def kernel(rpn_box_preds, rpn_cls_preds, batch_size, w1, g1, be1, rm1, rv1,
           w2, g2, be2, rm2, rv2, wb, bb, wr, br):
  bsz, n, _ = rpn_box_preds.shape
  npad = ((n + 127) // 128) * 128

  cls_t = jnp.transpose(rpn_cls_preds, (2, 0, 1))

  eps = 1e-5
  s1 = g1 * lax.rsqrt(rv1 + eps)
  wt1 = jnp.transpose(w1 * s1[:, None, None], (2, 1, 0))
  w1c = jnp.zeros((3, 16, 32), jnp.float32)
  w1c = w1c.at[:, 0:7].set(wt1[:, 0:7]).at[:, 8:11].set(wt1[:, 7:10])
  w1c = w1c.reshape(48, 32)
  b1c = be1 - rm1 * s1
  s2 = g2 * lax.rsqrt(rv2 + eps)
  w2c = jnp.transpose(w2 * s2[:, None, None], (2, 1, 0)).reshape(96, 64)
  b2c = be2 - rm2 * s2
  wb2 = wb[:, :, 0].T
  wr2 = wr[:, :, 0].T

  nc = 2000
  comb = pl.pallas_call(
      _pack_body,
      grid=(bsz, n // nc),
      in_specs=[
          pl.BlockSpec((1, nc, 7), lambda b, c: (b, c, 0)),
          pl.BlockSpec((1, nc, 3), lambda b, c: (b, c, 0)),
      ],
      out_specs=pl.BlockSpec((1, nc, 16), lambda b, c: (b, c, 0)),
      out_shape=jax.ShapeDtypeStruct((bsz, n, 16), jnp.float32),
  )(rpn_box_preds, rpn_cls_preds)

  keys = pl.pallas_call(
      functools.partial(_keys_body, n),
      out_shape=jax.ShapeDtypeStruct((bsz, npad), jnp.int32),
  )(cls_t)

  mesh = plsc.VectorSubcoreMesh(core_axis_name="c", subcore_axis_name="s",
                                num_cores=2, num_subcores=16)
  comb_sel = pl.kernel(
      functools.partial(_sc_body, n, npad),
      out_type=jax.ShapeDtypeStruct((bsz, SEL, 16), jnp.float32),
      mesh=mesh,
      compiler_params=pltpu.CompilerParams(needs_layout_passes=False,
                                           use_tc_tiling_on_sc=False),
      scratch_types=[
          pltpu.VMEM((npad,), jnp.int32),
          pltpu.VMEM((npad,), jnp.int32),
          pltpu.VMEM((npad,), jnp.int32),
          pltpu.VMEM((npad,), jnp.int32),
          pltpu.VMEM((RADIX, 4 * LANES), jnp.int32),
          pltpu.VMEM((32, 128), jnp.int32),
          pltpu.VMEM((1024, 16), jnp.float32),
          pltpu.SemaphoreType.DMA,
      ],
  )(keys, comb)

  iou_bin, iou_res = pl.pallas_call(
      _head_body,
      grid=(bsz,),
      in_specs=[
          pl.BlockSpec((1, SEL, 16), lambda i: (i, 0, 0)),
          pl.BlockSpec((48, 32), lambda i: (0, 0)),
          pl.BlockSpec((32,), lambda i: (0,)),
          pl.BlockSpec((96, 64), lambda i: (0, 0)),
          pl.BlockSpec((64,), lambda i: (0,)),
          pl.BlockSpec((64, 5), lambda i: (0, 0)),
          pl.BlockSpec((5,), lambda i: (0,)),
          pl.BlockSpec((64, 1), lambda i: (0, 0)),
          pl.BlockSpec((1,), lambda i: (0,)),
      ],
      out_specs=[
          pl.BlockSpec((1, SEL, 5), lambda i: (i, 0, 0)),
          pl.BlockSpec((1, SEL, 1), lambda i: (i, 0, 0)),
      ],
      out_shape=[
          jax.ShapeDtypeStruct((bsz, SEL, 5), jnp.float32),
          jax.ShapeDtypeStruct((bsz, SEL, 1), jnp.float32),
      ],
  )(comb_sel, w1c, b1c, w2c, b2c, wb2, bb, wr2, br)

  box_sel = comb_sel[:, :, 0:7]
  cls_sel = comb_sel[:, :, 8:11]
  return (iou_bin, iou_res, box_sel, cls_sel)


# speedup vs baseline: 1.4119x; 1.0235x over previous
import functools

import jax
import jax.numpy as jnp
from jax import lax
from jax.experimental import pallas as pl
from jax.experimental.pallas import tpu as pltpu
from jax.experimental.pallas import tpu_sc as plsc

SEL = 4096
LANES = 16
RADIX = 256


def _pack_body(box_ref, cls_ref, comb_ref):
  box = box_ref[0]
  cls = cls_ref[0]
  nc = box.shape[0]
  comb_ref[0] = jnp.concatenate(
      [box, jnp.zeros((nc, 1), jnp.float32), cls,
       jnp.zeros((nc, 5), jnp.float32)], axis=1)


def _keys_body(n, cls_ref, keys_ref):
  m = jnp.max(cls_ref[...], axis=0)
  bits = lax.bitcast_convert_type(m, jnp.int32)
  ck = jnp.where(bits < 0, bits, ~(bits | jnp.int32(-2147483648)))
  keys_ref[:, :n] = ck
  keys_ref[:, n:] = jnp.full(
      (keys_ref.shape[0], keys_ref.shape[1] - n), -1, jnp.int32)


def _sc_body(n, npad, keys_hbm, comb_hbm, out_hbm,
             keys_a, idx_a, keys_b, idx_b, hist, idx2d, rows, sem):
  b = lax.axis_index("s") * 2 + lax.axis_index("c")
  nb = keys_hbm.shape[0]
  nstream = 4
  vl = nstream * LANES
  chunk = npad // vl
  iota = lax.iota(jnp.int32, LANES)
  lane_bases = [(s * LANES + iota) * chunk for s in range(nstream)]
  lane_cols = [s * LANES + iota for s in range(nstream)]
  ones = jnp.ones((LANES,), jnp.int32)

  @pl.when(b < nb)
  def _():
    pltpu.sync_copy(keys_hbm.at[b], keys_a)

    @pl.loop(0, npad // LANES)
    def _(t):
      idx_a[pl.ds(t * LANES, LANES)] = t * LANES + iota

    for p, (ks, vs, kd, vd) in enumerate((
        (keys_a, idx_a, keys_b, idx_b),
        (keys_b, idx_b, keys_a, idx_a),
        (keys_a, idx_a, keys_b, idx_b),
        (keys_b, idx_b, None, None),
    )):
      shift = 8 * p

      @pl.loop(0, RADIX)
      def _(d):
        for s in range(nstream):
          hist[d, pl.ds(s * LANES, LANES)] = jnp.zeros((LANES,), jnp.int32)

      @pl.loop(0, chunk)
      def _(t):
        for s in range(nstream):
          k = plsc.load_gather(ks, [lane_bases[s] + t])
          d = lax.shift_right_logical(k, shift) & 0xFF
          plsc.addupdate_scatter(hist, [d, lane_cols[s]], ones)

      @pl.loop(0, RADIX, init_carry=jnp.int32(0))
      def _(d, run):
        for s in range(nstream):
          v = hist[d, pl.ds(s * LANES, LANES)]
          inc = plsc.cumsum(v)
          hist[d, pl.ds(s * LANES, LANES)] = (inc - v) + run
          run = run + jnp.sum(v)
        return run

      if kd is not None:
        @pl.loop(0, chunk)
        def _(t):
          for s in range(nstream):
            g = lane_bases[s] + t
            k = plsc.load_gather(ks, [g])
            v = plsc.load_gather(vs, [g])
            d = lax.shift_right_logical(k, shift) & 0xFF
            pos = plsc.load_gather(hist, [d, lane_cols[s]])
            plsc.store_scatter(kd, [pos], k)
            plsc.store_scatter(vd, [pos], v)
            plsc.store_scatter(hist, [d, lane_cols[s]], pos + 1)
      else:
        @pl.loop(0, chunk)
        def _(t):
          for s in range(nstream):
            g = lane_bases[s] + t
            k = plsc.load_gather(ks, [g])
            v = plsc.load_gather(vs, [g])
            d = lax.shift_right_logical(k, shift) & 0xFF
            pos = plsc.load_gather(hist, [d, lane_cols[s]])
            plsc.store_scatter(idx2d, [lax.shift_right_logical(pos, 7),
                                       pos & 127], v, mask=pos < SEL)
            plsc.store_scatter(hist, [d, lane_cols[s]], pos + 1)

    for q in range(4):
      descs = []
      for j in range(8):
        descs.append(pltpu.async_copy(
            comb_hbm.at[b].at[idx2d.at[8 * q + j]],
            rows.at[pl.ds(j * 128, 128)], sem))
      for dsc in descs:
        dsc.wait()
      pltpu.sync_copy(rows, out_hbm.at[b].at[pl.ds(q * 1024, 1024)])


def _mm(x, w):
  return lax.dot_general(x, w, (((1,), (0,)), ((), ())),
                         preferred_element_type=jnp.float32)


def _head_body(comb_ref, w1_ref, b1_ref, w2_ref, b2_ref, wb_ref, bb_ref,
               wr_ref, br_ref, bin_ref, res_ref, boxo_ref, clso_ref):
  x = comb_ref[0]
  boxo_ref[0] = x[:, 0:7]
  clso_ref[0] = x[:, 8:11]
  z = jnp.zeros((1, x.shape[1]), jnp.float32)
  xd = jnp.concatenate([z, x[:-1, :]], axis=0)
  xu = jnp.concatenate([x[1:, :], z], axis=0)
  w1 = w1_ref[...]
  h1 = _mm(xd, w1[0:16]) + _mm(x, w1[16:32]) + _mm(xu, w1[32:48])
  h1 = jnp.maximum(h1 + b1_ref[...], 0.0)
  z1 = jnp.zeros((1, h1.shape[1]), jnp.float32)
  h1d = jnp.concatenate([z1, h1[:-1, :]], axis=0)
  h1u = jnp.concatenate([h1[1:, :], z1], axis=0)
  w2 = w2_ref[...]
  h2 = _mm(h1d, w2[0:32]) + _mm(h1, w2[32:64]) + _mm(h1u, w2[64:96])
  h2 = jnp.maximum(h2 + b2_ref[...], 0.0)
  bin_ref[0] = _mm(h2, wb_ref[...]) + bb_ref[...]
  res_ref[0] = _mm(h2, wr_ref[...]) + br_ref[...]


def kernel(rpn_box_preds, rpn_cls_preds, batch_size, w1, g1, be1, rm1, rv1,
           w2, g2, be2, rm2, rv2, wb, bb, wr, br):
  bsz, n, _ = rpn_box_preds.shape
  npad = ((n + 127) // 128) * 128

  cls_t = jnp.transpose(rpn_cls_preds, (2, 0, 1))

  eps = 1e-5
  s1 = g1 * lax.rsqrt(rv1 + eps)
  wt1 = jnp.transpose(w1 * s1[:, None, None], (2, 1, 0))
  w1c = jnp.zeros((3, 16, 32), jnp.float32)
  w1c = w1c.at[:, 0:7].set(wt1[:, 0:7]).at[:, 8:11].set(wt1[:, 7:10])
  w1c = w1c.reshape(48, 32)
  b1c = be1 - rm1 * s1
  s2 = g2 * lax.rsqrt(rv2 + eps)
  w2c = jnp.transpose(w2 * s2[:, None, None], (2, 1, 0)).reshape(96, 64)
  b2c = be2 - rm2 * s2
  wb2 = wb[:, :, 0].T
  wr2 = wr[:, :, 0].T

  nc = 2000
  comb = pl.pallas_call(
      _pack_body,
      grid=(bsz, n // nc),
      in_specs=[
          pl.BlockSpec((1, nc, 7), lambda b, c: (b, c, 0)),
          pl.BlockSpec((1, nc, 3), lambda b, c: (b, c, 0)),
      ],
      out_specs=pl.BlockSpec((1, nc, 16), lambda b, c: (b, c, 0)),
      out_shape=jax.ShapeDtypeStruct((bsz, n, 16), jnp.float32),
  )(rpn_box_preds, rpn_cls_preds)

  keys = pl.pallas_call(
      functools.partial(_keys_body, n),
      out_shape=jax.ShapeDtypeStruct((bsz, npad), jnp.int32),
  )(cls_t)

  mesh = plsc.VectorSubcoreMesh(core_axis_name="c", subcore_axis_name="s",
                                num_cores=2, num_subcores=16)
  comb_sel = pl.kernel(
      functools.partial(_sc_body, n, npad),
      out_type=jax.ShapeDtypeStruct((bsz, SEL, 16), jnp.float32),
      mesh=mesh,
      compiler_params=pltpu.CompilerParams(needs_layout_passes=False,
                                           use_tc_tiling_on_sc=False),
      scratch_types=[
          pltpu.VMEM((npad,), jnp.int32),
          pltpu.VMEM((npad,), jnp.int32),
          pltpu.VMEM((npad,), jnp.int32),
          pltpu.VMEM((npad,), jnp.int32),
          pltpu.VMEM((RADIX, 4 * LANES), jnp.int32),
          pltpu.VMEM((32, 128), jnp.int32),
          pltpu.VMEM((1024, 16), jnp.float32),
          pltpu.SemaphoreType.DMA,
      ],
  )(keys, comb)

  head_out = pl.pallas_call(
      _head_body,
      grid=(bsz,),
      in_specs=[
          pl.BlockSpec((1, SEL, 16), lambda i: (i, 0, 0)),
          pl.BlockSpec((48, 32), lambda i: (0, 0)),
          pl.BlockSpec((32,), lambda i: (0,)),
          pl.BlockSpec((96, 64), lambda i: (0, 0)),
          pl.BlockSpec((64,), lambda i: (0,)),
          pl.BlockSpec((64, 5), lambda i: (0, 0)),
          pl.BlockSpec((5,), lambda i: (0,)),
          pl.BlockSpec((64, 1), lambda i: (0, 0)),
          pl.BlockSpec((1,), lambda i: (0,)),
      ],
      out_specs=[
          pl.BlockSpec((1, SEL, 5), lambda i: (i, 0, 0)),
          pl.BlockSpec((1, SEL, 1), lambda i: (i, 0, 0)),
          pl.BlockSpec((1, SEL, 7), lambda i: (i, 0, 0)),
          pl.BlockSpec((1, SEL, 3), lambda i: (i, 0, 0)),
      ],
      out_shape=[
          jax.ShapeDtypeStruct((bsz, SEL, 5), jnp.float32),
          jax.ShapeDtypeStruct((bsz, SEL, 1), jnp.float32),
          jax.ShapeDtypeStruct((bsz, SEL, 7), jnp.float32),
          jax.ShapeDtypeStruct((bsz, SEL, 3), jnp.float32),
      ],
  )(comb_sel, w1c, b1c, w2c, b2c, wb2, bb, wr2, br)

  iou_bin, iou_res, box_sel, cls_sel = head_out
  return (iou_bin, iou_res, box_sel, cls_sel)
